# Initial kernel scaffold; baseline (speedup 1.0000x reference)
import functools

import jax
import jax.numpy as jnp
from jax import lax
from jax.experimental import pallas as pl
from jax.experimental.pallas import tpu as pltpu
from jax.experimental.pallas import tpu_sc as plsc

_N = 10000
_F = 128
_E = 320000
_BLK = 1000
_NROW = _N // _BLK
_NW = 32
_CH = 128
_NCH = _E // _CH
_MAXIT = -(-_NCH // _NW)
_RPT = _N // 16


def _tc_proj_body(x_ref, w_ref, asrc_ref, adst_ref, h_ref, es_ref, ed_ref):
    h = jnp.dot(x_ref[...], w_ref[...], preferred_element_type=jnp.float32)
    h_ref[...] = h
    es_ref[...] = jnp.sum(h * asrc_ref[...], axis=1).reshape(1, _BLK)
    ed_ref[...] = jnp.sum(h * adst_ref[...], axis=1).reshape(1, _BLK)


def _tc_proj(x, w, a_src, a_dst):
    return pl.pallas_call(
        _tc_proj_body,
        grid=(_NROW,),
        in_specs=[
            pl.BlockSpec((_BLK, _F), lambda i: (i, 0)),
            pl.BlockSpec((_F, _F), lambda i: (0, 0)),
            pl.BlockSpec((1, _F), lambda i: (0, 0)),
            pl.BlockSpec((1, _F), lambda i: (0, 0)),
        ],
        out_specs=[
            pl.BlockSpec((_BLK, _F), lambda i: (i, 0)),
            pl.BlockSpec((1, _BLK), lambda i: (i, 0)),
            pl.BlockSpec((1, _BLK), lambda i: (i, 0)),
        ],
        out_shape=[
            jax.ShapeDtypeStruct((_N, _F), jnp.float32),
            jax.ShapeDtypeStruct((_NROW, _BLK), jnp.float32),
            jax.ShapeDtypeStruct((_NROW, _BLK), jnp.float32),
        ],
    )(x, w, a_src, a_dst)


def _tc_comb_body(acc_ref, den_ref, w_ref, asrc_ref, adst_ref,
                  h_ref, es_ref, ed_ref):
    a = acc_ref[0] + acc_ref[1]
    d = den_ref[0] + den_ref[1]
    g = a / (d[:, None] + 1e-16)
    g = jnp.where(g > 0, g, jnp.expm1(g))
    h = jnp.dot(g, w_ref[...], preferred_element_type=jnp.float32)
    h_ref[...] = h
    es_ref[...] = jnp.sum(h * asrc_ref[...], axis=1).reshape(1, _BLK)
    ed_ref[...] = jnp.sum(h * adst_ref[...], axis=1).reshape(1, _BLK)


def _tc_comb(acc, den, w, a_src, a_dst):
    return pl.pallas_call(
        _tc_comb_body,
        grid=(_NROW,),
        in_specs=[
            pl.BlockSpec((2, _BLK, _F), lambda i: (0, i, 0)),
            pl.BlockSpec((2, _BLK), lambda i: (0, i)),
            pl.BlockSpec((_F, _F), lambda i: (0, 0)),
            pl.BlockSpec((1, _F), lambda i: (0, 0)),
            pl.BlockSpec((1, _F), lambda i: (0, 0)),
        ],
        out_specs=[
            pl.BlockSpec((_BLK, _F), lambda i: (i, 0)),
            pl.BlockSpec((1, _BLK), lambda i: (i, 0)),
            pl.BlockSpec((1, _BLK), lambda i: (i, 0)),
        ],
        out_shape=[
            jax.ShapeDtypeStruct((_N, _F), jnp.float32),
            jax.ShapeDtypeStruct((_NROW, _BLK), jnp.float32),
            jax.ShapeDtypeStruct((_NROW, _BLK), jnp.float32),
        ],
    )(acc, den, w, a_src, a_dst)


def _tc_final_body(acc_ref, den_ref, wl_ref, bl_ref, a_ref, b_ref):
    a = acc_ref[0] + acc_ref[1]
    d = den_ref[0] + den_ref[1]
    h2 = a / (d[:, None] + 1e-16)
    a_ref[...] = (jnp.dot(h2, wl_ref[0], preferred_element_type=jnp.float32)
                  + bl_ref[...])
    b_ref[...] = jnp.dot(h2, wl_ref[1], preferred_element_type=jnp.float32)


def _tc_final(acc, den, wl, bl):
    return pl.pallas_call(
        _tc_final_body,
        grid=(_NROW,),
        in_specs=[
            pl.BlockSpec((2, _BLK, _F), lambda i: (0, i, 0)),
            pl.BlockSpec((2, _BLK), lambda i: (0, i)),
            pl.BlockSpec((2, _F, _F), lambda i: (0, 0, 0)),
            pl.BlockSpec((1, _F), lambda i: (0, 0)),
        ],
        out_specs=[
            pl.BlockSpec((_BLK, _F), lambda i: (i, 0)),
            pl.BlockSpec((_BLK, _F), lambda i: (i, 0)),
        ],
        out_shape=[
            jax.ShapeDtypeStruct((_N, _F), jnp.float32),
            jax.ShapeDtypeStruct((_N, _F), jnp.float32),
        ],
    )(acc, den, wl, bl)


_sc_mesh = plsc.VectorSubcoreMesh(core_axis_name="c", subcore_axis_name="s")


@functools.partial(
    pl.kernel,
    mesh=_sc_mesh,
    out_type=[
        jax.ShapeDtypeStruct((2, _N, _F), jnp.float32),
        jax.ShapeDtypeStruct((2, _N), jnp.float32),
    ],
    scratch_types=[
        pltpu.VMEM((_CH,), jnp.int32),
        pltpu.VMEM((_CH,), jnp.int32),
        pltpu.VMEM((_CH,), jnp.float32),
        pltpu.VMEM((_CH,), jnp.float32),
        pltpu.VMEM((_CH,), jnp.float32),
        pltpu.VMEM((_CH, _F), jnp.float32),
        pltpu.VMEM((_N,), jnp.float32),
        pltpu.VMEM_SHARED((_N, _F), jnp.float32),
        pltpu.VMEM_SHARED((_N,), jnp.float32),
        pltpu.SemaphoreType.DMA,
    ],
)
def _sc_gat(h_hbm, es_hbm, ed_hbm, src_hbm, dst_hbm, acc_out, den_out,
            idx_s, idx_d, esg, edg, pbuf, rows, den_l, acc_s, den_s, sem):
    c = lax.axis_index("c")
    s = lax.axis_index("s")
    w = s * 2 + c
    z16 = jnp.zeros((16,), jnp.float32)

    def _zero_row(r, carry):
        for j in range(8):
            rows[r, pl.ds(16 * j, 16)] = z16
        return carry

    lax.fori_loop(0, _CH, _zero_row, 0)

    def _zero_den(i, carry):
        den_l[pl.ds(16 * i, 16)] = z16
        return carry

    lax.fori_loop(0, _N // 16, _zero_den, 0)

    base_r = s * _RPT
    for off in (0, _CH, 2 * _CH, 3 * _CH):
        pltpu.sync_copy(rows, acc_s.at[pl.ds(base_r + off, _CH)])
    tail = _RPT - 4 * _CH
    pltpu.sync_copy(rows.at[pl.ds(0, tail)],
                    acc_s.at[pl.ds(base_r + 4 * _CH, tail)])

    @pl.when(s == 0)
    def _():
        pltpu.sync_copy(den_l, den_s)

    plsc.subcore_barrier()

    def _chunk(i, carry):
        k = w + i * _NW

        @pl.when(k < _NCH)
        def _():
            base = k * _CH
            pltpu.sync_copy(src_hbm.at[pl.ds(base, _CH)], idx_s)
            pltpu.sync_copy(dst_hbm.at[pl.ds(base, _CH)], idx_d)
            cp = pltpu.async_copy(h_hbm.at[idx_s], rows, sem)
            pltpu.sync_copy(es_hbm.at[idx_s], esg)
            pltpu.sync_copy(ed_hbm.at[idx_d], edg)
            for v in range(_CH // 16):
                sl = pl.ds(16 * v, 16)
                t = esg[sl] + edg[sl]
                l = jnp.maximum(t, t * 0.2)
                l = jnp.minimum(l, 80.0)
                p = jnp.exp(l)
                pbuf[sl] = p
                plsc.addupdate_scatter(den_l, [idx_d[sl]], p)
            cp.wait()

            def _scale(r, cc):
                pv = lax.broadcast(pbuf[r], (16,))
                for j in range(8):
                    sj = pl.ds(16 * j, 16)
                    rows[r, sj] = rows[r, sj] * pv
                return cc

            lax.fori_loop(0, _CH, _scale, 0)
            pltpu.sync_copy(rows, acc_s.at[idx_d], add=True)

        return carry

    lax.fori_loop(0, _MAXIT, _chunk, 0)

    pltpu.sync_copy(den_l, den_s, add=True)
    plsc.subcore_barrier()

    pltpu.sync_copy(acc_s.at[pl.ds(base_r, _RPT)],
                    acc_out.at[c, pl.ds(base_r, _RPT)])

    @pl.when(s == 0)
    def _():
        pltpu.sync_copy(den_s, den_out.at[c])


@functools.partial(
    pl.kernel,
    mesh=_sc_mesh,
    out_type=jax.ShapeDtypeStruct((_E, _F), jnp.float32),
    scratch_types=[
        pltpu.VMEM((_CH,), jnp.int32),
        pltpu.VMEM((_CH,), jnp.int32),
        pltpu.VMEM((_CH, _F), jnp.float32),
        pltpu.VMEM((_CH, _F), jnp.float32),
        pltpu.SemaphoreType.DMA,
        pltpu.SemaphoreType.DMA,
    ],
)
def _sc_edge(a_hbm, b_hbm, src_hbm, dst_hbm, out_hbm,
             idx_s, idx_d, ra, rb, sem_a, sem_b):
    c = lax.axis_index("c")
    s = lax.axis_index("s")
    w = s * 2 + c

    def _chunk(i, carry):
        k = w + i * _NW

        @pl.when(k < _NCH)
        def _():
            base = k * _CH
            pltpu.sync_copy(src_hbm.at[pl.ds(base, _CH)], idx_s)
            pltpu.sync_copy(dst_hbm.at[pl.ds(base, _CH)], idx_d)
            ca = pltpu.async_copy(a_hbm.at[idx_s], ra, sem_a)
            cb = pltpu.async_copy(b_hbm.at[idx_d], rb, sem_b)
            ca.wait()
            cb.wait()

            def _addrow(r, cc):
                for j in range(8):
                    sj = pl.ds(16 * j, 16)
                    ra[r, sj] = ra[r, sj] + rb[r, sj]
                return cc

            lax.fori_loop(0, _CH, _addrow, 0)
            pltpu.sync_copy(ra, out_hbm.at[pl.ds(base, _CH)])

        return carry

    lax.fori_loop(0, _MAXIT, _chunk, 0)


def kernel(x, edge_index, W1, a1_src, a1_dst, W2, a2_src, a2_dst, Wl, bl):
    src = edge_index[0]
    dst = edge_index[1]
    h1p, es1, ed1 = _tc_proj(x, W1, a1_src.reshape(1, _F),
                             a1_dst.reshape(1, _F))
    acc1, den1 = _sc_gat(h1p, es1.reshape(_N), ed1.reshape(_N), src, dst)
    h2p, es2, ed2 = _tc_comb(acc1, den1, W2, a2_src.reshape(1, _F),
                             a2_dst.reshape(1, _F))
    acc2, den2 = _sc_gat(h2p, es2.reshape(_N), ed2.reshape(_N), src, dst)
    a_n, b_n = _tc_final(acc2, den2, Wl.reshape(2, _F, _F), bl.reshape(1, _F))
    return _sc_edge(a_n, b_n, src, dst)


# speedup vs baseline: 13.3632x; 13.3632x over previous
import functools

import jax
import jax.numpy as jnp
from jax import lax
from jax.experimental import pallas as pl
from jax.experimental.pallas import tpu as pltpu
from jax.experimental.pallas import tpu_sc as plsc

_N = 10000
_F = 128
_E = 320000
_BLK = 1000
_NROW = _N // _BLK
_NW = 32
_CH = 128
_NCH = _E // _CH
_MAXIT = -(-_NCH // _NW)
_RPT = _N // 16


def _tc_proj_body(x_ref, w_ref, asrc_ref, adst_ref, h_ref, es_ref, ed_ref):
    h = jnp.dot(x_ref[...], w_ref[...], preferred_element_type=jnp.float32)
    h_ref[...] = h
    es_ref[...] = jnp.sum(h * asrc_ref[...], axis=1).reshape(_BLK, 1)
    ed_ref[...] = jnp.sum(h * adst_ref[...], axis=1).reshape(_BLK, 1)


def _tc_proj(x, w, a_src, a_dst):
    return pl.pallas_call(
        _tc_proj_body,
        grid=(_NROW,),
        in_specs=[
            pl.BlockSpec((_BLK, _F), lambda i: (i, 0)),
            pl.BlockSpec((_F, _F), lambda i: (0, 0)),
            pl.BlockSpec((1, _F), lambda i: (0, 0)),
            pl.BlockSpec((1, _F), lambda i: (0, 0)),
        ],
        out_specs=[
            pl.BlockSpec((_BLK, _F), lambda i: (i, 0)),
            pl.BlockSpec((_BLK, 1), lambda i: (i, 0)),
            pl.BlockSpec((_BLK, 1), lambda i: (i, 0)),
        ],
        out_shape=[
            jax.ShapeDtypeStruct((_N, _F), jnp.float32),
            jax.ShapeDtypeStruct((_N, 1), jnp.float32),
            jax.ShapeDtypeStruct((_N, 1), jnp.float32),
        ],
    )(x, w, a_src, a_dst)


def _tc_comb_body(acc_ref, den_ref, w_ref, asrc_ref, adst_ref,
                  h_ref, es_ref, ed_ref):
    a = acc_ref[0] + acc_ref[1]
    d = jnp.sum(den_ref[...], axis=0)
    g = a / (d + 1e-16)
    g = jnp.where(g > 0, g, jnp.exp(g) - 1.0)
    h = jnp.dot(g, w_ref[...], preferred_element_type=jnp.float32)
    h_ref[...] = h
    es_ref[...] = jnp.sum(h * asrc_ref[...], axis=1).reshape(_BLK, 1)
    ed_ref[...] = jnp.sum(h * adst_ref[...], axis=1).reshape(_BLK, 1)


def _tc_comb(acc, den, w, a_src, a_dst):
    return pl.pallas_call(
        _tc_comb_body,
        grid=(_NROW,),
        in_specs=[
            pl.BlockSpec((2, _BLK, _F), lambda i: (0, i, 0)),
            pl.BlockSpec((_NW, _BLK, 1), lambda i: (0, i, 0)),
            pl.BlockSpec((_F, _F), lambda i: (0, 0)),
            pl.BlockSpec((1, _F), lambda i: (0, 0)),
            pl.BlockSpec((1, _F), lambda i: (0, 0)),
        ],
        out_specs=[
            pl.BlockSpec((_BLK, _F), lambda i: (i, 0)),
            pl.BlockSpec((_BLK, 1), lambda i: (i, 0)),
            pl.BlockSpec((_BLK, 1), lambda i: (i, 0)),
        ],
        out_shape=[
            jax.ShapeDtypeStruct((_N, _F), jnp.float32),
            jax.ShapeDtypeStruct((_N, 1), jnp.float32),
            jax.ShapeDtypeStruct((_N, 1), jnp.float32),
        ],
    )(acc, den, w, a_src, a_dst)


def _tc_final_body(acc_ref, den_ref, wl_ref, bl_ref, a_ref, b_ref):
    a = acc_ref[0] + acc_ref[1]
    d = jnp.sum(den_ref[...], axis=0)
    h2 = a / (d + 1e-16)
    a_ref[...] = (jnp.dot(h2, wl_ref[0], preferred_element_type=jnp.float32)
                  + bl_ref[...])
    b_ref[...] = jnp.dot(h2, wl_ref[1], preferred_element_type=jnp.float32)


def _tc_final(acc, den, wl, bl):
    return pl.pallas_call(
        _tc_final_body,
        grid=(_NROW,),
        in_specs=[
            pl.BlockSpec((2, _BLK, _F), lambda i: (0, i, 0)),
            pl.BlockSpec((_NW, _BLK, 1), lambda i: (0, i, 0)),
            pl.BlockSpec((2, _F, _F), lambda i: (0, 0, 0)),
            pl.BlockSpec((1, _F), lambda i: (0, 0)),
        ],
        out_specs=[
            pl.BlockSpec((_BLK, _F), lambda i: (i, 0)),
            pl.BlockSpec((_BLK, _F), lambda i: (i, 0)),
        ],
        out_shape=[
            jax.ShapeDtypeStruct((_N, _F), jnp.float32),
            jax.ShapeDtypeStruct((_N, _F), jnp.float32),
        ],
    )(acc, den, wl, bl)


_sc_mesh = plsc.VectorSubcoreMesh(core_axis_name="c", subcore_axis_name="s")


@functools.partial(
    pl.kernel,
    mesh=_sc_mesh,
    out_type=[
        jax.ShapeDtypeStruct((2, _N, _F), jnp.float32),
        jax.ShapeDtypeStruct((_NW * _N,), jnp.float32),
    ],
    scratch_types=[
        pltpu.VMEM((_CH,), jnp.int32),
        pltpu.VMEM((_CH,), jnp.int32),
        pltpu.VMEM((_CH,), jnp.float32),
        pltpu.VMEM((_CH,), jnp.float32),
        pltpu.VMEM((_CH,), jnp.float32),
        pltpu.VMEM((_CH, _F), jnp.float32),
        pltpu.VMEM((_N,), jnp.float32),
        pltpu.VMEM_SHARED((_N, _F), jnp.float32),
        pltpu.SemaphoreType.DMA,
    ],
    compiler_params=pltpu.CompilerParams(needs_layout_passes=False),
)
def _sc_gat(h_hbm, es_hbm, ed_hbm, src_hbm, dst_hbm, acc_out, den_out,
            idx_s, idx_d, esg, edg, pbuf, rows, den_l, acc_s, sem):
    c = lax.axis_index("c")
    s = lax.axis_index("s")
    w = s * 2 + c
    z16 = jnp.zeros((16,), jnp.float32)

    def _zero_row(r, carry):
        for j in range(8):
            rows[r, pl.ds(16 * j, 16)] = z16
        return carry

    lax.fori_loop(0, _CH, _zero_row, 0)

    def _zero_den(i, carry):
        den_l[pl.ds(16 * i, 16)] = z16
        return carry

    lax.fori_loop(0, _N // 16, _zero_den, 0)

    base_r = s * 640

    @pl.when(s < 15)
    def _():
        for off in (0, 128, 256, 384, 512):
            pltpu.sync_copy(rows, acc_s.at[pl.ds(base_r + off, _CH)])

    @pl.when(s == 15)
    def _():
        for off in (0, 128, 256):
            pltpu.sync_copy(rows, acc_s.at[pl.ds(base_r + off, _CH)])
        pltpu.sync_copy(rows.at[pl.ds(0, 16)],
                        acc_s.at[pl.ds(base_r + 384, 16)])

    plsc.subcore_barrier()

    def _chunk(i, carry):
        k = w + i * _NW

        @pl.when(k < _NCH)
        def _():
            base = k * _CH
            pltpu.sync_copy(src_hbm.at[pl.ds(base, _CH)], idx_s)
            pltpu.sync_copy(dst_hbm.at[pl.ds(base, _CH)], idx_d)
            cp = pltpu.async_copy(h_hbm.at[idx_s], rows, sem)
            pltpu.sync_copy(es_hbm.at[idx_s], esg)
            pltpu.sync_copy(ed_hbm.at[idx_d], edg)
            for v in range(_CH // 16):
                sl = pl.ds(16 * v, 16)
                t = esg[sl] + edg[sl]
                l = jnp.maximum(t, t * 0.2)
                l = jnp.minimum(l, 80.0)
                p = jnp.exp(l)
                pbuf[sl] = p
                plsc.addupdate_scatter(den_l, [idx_d[sl]], p)
            cp.wait()

            def _scale(r, cc):
                pv = plsc.load_gather(pbuf, [lax.broadcast(r, (16,))])
                for j in range(8):
                    sj = pl.ds(16 * j, 16)
                    rows[r, sj] = rows[r, sj] * pv
                return cc

            lax.fori_loop(0, _CH, _scale, 0)
            pltpu.sync_copy(rows, acc_s.at[idx_d], add=True)

        return carry

    lax.fori_loop(0, _MAXIT, _chunk, 0)

    pltpu.sync_copy(den_l, den_out.at[pl.ds(w * _N, _N)])
    plsc.subcore_barrier()

    @pl.when(s < 15)
    def _():
        pltpu.sync_copy(acc_s.at[pl.ds(base_r, 640)],
                        acc_out.at[c, pl.ds(base_r, 640)])

    @pl.when(s == 15)
    def _():
        pltpu.sync_copy(acc_s.at[pl.ds(base_r, 400)],
                        acc_out.at[c, pl.ds(base_r, 400)])


@functools.partial(
    pl.kernel,
    mesh=_sc_mesh,
    out_type=jax.ShapeDtypeStruct((_E, _F), jnp.float32),
    scratch_types=[
        pltpu.VMEM((_CH,), jnp.int32),
        pltpu.VMEM((_CH,), jnp.int32),
        pltpu.VMEM((_CH, _F), jnp.float32),
        pltpu.VMEM((_CH, _F), jnp.float32),
        pltpu.SemaphoreType.DMA,
        pltpu.SemaphoreType.DMA,
    ],
    compiler_params=pltpu.CompilerParams(needs_layout_passes=False),
)
def _sc_edge(a_hbm, b_hbm, src_hbm, dst_hbm, out_hbm,
             idx_s, idx_d, ra, rb, sem_a, sem_b):
    c = lax.axis_index("c")
    s = lax.axis_index("s")
    w = s * 2 + c

    def _chunk(i, carry):
        k = w + i * _NW

        @pl.when(k < _NCH)
        def _():
            base = k * _CH
            pltpu.sync_copy(src_hbm.at[pl.ds(base, _CH)], idx_s)
            pltpu.sync_copy(dst_hbm.at[pl.ds(base, _CH)], idx_d)
            ca = pltpu.async_copy(a_hbm.at[idx_s], ra, sem_a)
            cb = pltpu.async_copy(b_hbm.at[idx_d], rb, sem_b)
            ca.wait()
            cb.wait()

            def _addrow(r, cc):
                for j in range(8):
                    sj = pl.ds(16 * j, 16)
                    ra[r, sj] = ra[r, sj] + rb[r, sj]
                return cc

            lax.fori_loop(0, _CH, _addrow, 0)
            pltpu.sync_copy(ra, out_hbm.at[pl.ds(base, _CH)])

        return carry

    lax.fori_loop(0, _MAXIT, _chunk, 0)


def kernel(x, edge_index, W1, a1_src, a1_dst, W2, a2_src, a2_dst, Wl, bl):
    src = edge_index[0]
    dst = edge_index[1]
    h1p, es1, ed1 = _tc_proj(x, W1, a1_src.reshape(1, _F),
                             a1_dst.reshape(1, _F))
    acc1, den1 = _sc_gat(h1p, es1.reshape(_N), ed1.reshape(_N), src, dst)
    den1 = den1.reshape(_NW, _N, 1)
    h2p, es2, ed2 = _tc_comb(acc1, den1, W2, a2_src.reshape(1, _F),
                             a2_dst.reshape(1, _F))
    acc2, den2 = _sc_gat(h2p, es2.reshape(_N), ed2.reshape(_N), src, dst)
    den2 = den2.reshape(_NW, _N, 1)
    a_n, b_n = _tc_final(acc2, den2, Wl.reshape(2, _F, _F), bl.reshape(1, _F))
    return _sc_edge(a_n, b_n, src, dst)
